# baseline (speedup 1.0000x reference)
import functools

import jax
import jax.numpy as jnp
from jax import lax
from jax.experimental import pallas as pl
from jax.experimental.pallas import tpu as pltpu
from jax.experimental.pallas import tpu_sc as plsc

_VOCAB = 1000000
_EMB = 64
_BATCH = 4096
_SEQ = 200
_CLASSES = 5

_NC = 2
_NS = 16
_NW = _NC * _NS
_ROWS_PER_W = _BATCH // _NW
_CH = 104
_SEQ_PAD = 2 * _CH

_mesh = plsc.VectorSubcoreMesh(core_axis_name="c", subcore_axis_name="s")


@functools.partial(
    pl.kernel,
    mesh=_mesh,
    out_type=jax.ShapeDtypeStruct((_BATCH, _EMB), jnp.float32),
    scratch_types=[
        pltpu.VMEM((2 * _ROWS_PER_W, _CH), jnp.int32),
        pltpu.VMEM((2, _SEQ_PAD, _EMB), jnp.float32),
        pltpu.VMEM((_ROWS_PER_W, _EMB), jnp.float32),
        pltpu.SemaphoreType.DMA,
    ],
    compiler_params=pltpu.CompilerParams(use_tc_tiling_on_sc=False),
)
def _sc_pool(table_hbm, idx_hbm, out_hbm, idx_v, buf_v, pooled_v, sem):
    wid = lax.axis_index("s") * _NC + lax.axis_index("c")
    idx_base = wid * (2 * _ROWS_PER_W)

    pltpu.sync_copy(idx_hbm.at[pl.ds(idx_base, 2 * _ROWS_PER_W)], idx_v)

    def issue(row, slot):
        pltpu.async_copy(
            table_hbm.at[idx_v.at[2 * row]], buf_v.at[slot, pl.ds(0, _CH)], sem)
        pltpu.async_copy(
            table_hbm.at[idx_v.at[2 * row + 1]], buf_v.at[slot, pl.ds(_CH, _CH)],
            sem)

    issue(0, 0)

    def row_body(b, _):
        slot = lax.rem(b, 2)

        @pl.when(b + 1 < _ROWS_PER_W)
        def _():
            issue(b + 1, 1 - slot)

        pltpu.make_async_copy(
            table_hbm.at[idx_v.at[2 * b]], buf_v.at[slot, pl.ds(0, _CH)],
            sem).wait()
        pltpu.make_async_copy(
            table_hbm.at[idx_v.at[2 * b]], buf_v.at[slot, pl.ds(_CH, _CH)],
            sem).wait()

        def acc_body(s, accs):
            return tuple(
                accs[c] + buf_v[slot, s, pl.ds(c * 16, 16)] for c in range(4))

        zero = jnp.zeros((16,), jnp.float32)
        accs = lax.fori_loop(0, _SEQ, acc_body, (zero, zero, zero, zero),
                             unroll=4)
        for c in range(4):
            pooled_v[b, pl.ds(c * 16, 16)] = accs[c]
        return 0

    lax.fori_loop(0, _ROWS_PER_W, row_body, 0)
    pltpu.sync_copy(pooled_v, out_hbm.at[pl.ds(wid * _ROWS_PER_W, _ROWS_PER_W)])


def _mm_body(p_ref, w_ref, b_ref, o_ref):
    pooled = p_ref[...] * (1.0 / _SEQ)
    o_ref[...] = (
        jnp.dot(pooled, w_ref[...].T, preferred_element_type=jnp.float32)
        + b_ref[...])


def kernel(inputs, emb_table, W, b):
    idx = jnp.pad(inputs, ((0, 0), (0, _SEQ_PAD - _SEQ)))
    idx = idx.reshape(2 * _BATCH, _CH)
    sums = _sc_pool(emb_table, idx)
    out = pl.pallas_call(
        _mm_body,
        out_shape=jax.ShapeDtypeStruct((_BATCH, _CLASSES), jnp.float32),
    )(sums, W, b.reshape(1, _CLASSES))
    return out


# speedup vs baseline: 1.9618x; 1.9618x over previous
import functools

import jax
import jax.numpy as jnp
from jax import lax
from jax.experimental import pallas as pl
from jax.experimental.pallas import tpu as pltpu
from jax.experimental.pallas import tpu_sc as plsc

_VOCAB = 1000000
_EMB = 64
_BATCH = 4096
_SEQ = 200
_CLASSES = 5

_NC = 2
_NS = 16
_NW = _NC * _NS
_ROWS_PER_W = _BATCH // _NW
_CHA = 96
_CHB = _SEQ - _CHA
_NBUF = 4

_mesh = plsc.VectorSubcoreMesh(core_axis_name="c", subcore_axis_name="s")


@functools.partial(
    pl.kernel,
    mesh=_mesh,
    out_type=jax.ShapeDtypeStruct((_BATCH, _EMB), jnp.float32),
    scratch_types=[
        pltpu.VMEM((_ROWS_PER_W, _SEQ), jnp.int32),
        pltpu.VMEM((_NBUF, _SEQ, _EMB), jnp.float32),
        pltpu.VMEM((_ROWS_PER_W, _EMB), jnp.float32),
        pltpu.SemaphoreType.DMA,
    ],
    compiler_params=pltpu.CompilerParams(use_tc_tiling_on_sc=False),
)
def _sc_pool(table_hbm, idx_hbm, out_hbm, idx_v, buf_v, pooled_v, sem):
    wid = lax.axis_index("s") * _NC + lax.axis_index("c")
    row_base = wid * _ROWS_PER_W

    pltpu.sync_copy(idx_hbm.at[pl.ds(row_base, _ROWS_PER_W)], idx_v)

    def copies(row, slot):
        return (
            pltpu.make_async_copy(
                table_hbm.at[idx_v.at[row, pl.ds(0, _CHA)]],
                buf_v.at[slot, pl.ds(0, _CHA)], sem),
            pltpu.make_async_copy(
                table_hbm.at[idx_v.at[row, pl.ds(_CHA, _CHB)]],
                buf_v.at[slot, pl.ds(_CHA, _CHB)], sem),
        )

    def issue(row, slot):
        for c in copies(row, slot):
            c.start()

    for r in range(_NBUF - 1):
        issue(r, r)

    def row_body(b, _):
        slot = lax.rem(b, _NBUF)

        @pl.when(b + (_NBUF - 1) < _ROWS_PER_W)
        def _():
            issue(b + (_NBUF - 1), lax.rem(b + (_NBUF - 1), _NBUF))

        for c in copies(b, slot):
            c.wait()

        def acc_body(s, accs):
            return tuple(
                accs[c] + buf_v[slot, s, pl.ds(c * 16, 16)] for c in range(4))

        zero = jnp.zeros((16,), jnp.float32)
        accs = lax.fori_loop(0, _SEQ, acc_body, (zero, zero, zero, zero),
                             unroll=4)
        for c in range(4):
            pooled_v[b, pl.ds(c * 16, 16)] = accs[c]
        return 0

    lax.fori_loop(0, _ROWS_PER_W, row_body, 0)
    pltpu.sync_copy(pooled_v, out_hbm.at[pl.ds(row_base, _ROWS_PER_W)])


def _mm_body(p_ref, w_ref, b_ref, o_ref):
    pooled = p_ref[...] * (1.0 / _SEQ)
    o_ref[...] = (
        jnp.dot(pooled, w_ref[...].T, preferred_element_type=jnp.float32)
        + b_ref[...])


def kernel(inputs, emb_table, W, b):
    sums = _sc_pool(emb_table, inputs)
    out = pl.pallas_call(
        _mm_body,
        out_shape=jax.ShapeDtypeStruct((_BATCH, _CLASSES), jnp.float32),
    )(sums, W, b.reshape(1, _CLASSES))
    return out


# speedup vs baseline: 3.3536x; 1.7094x over previous
import functools

import jax
import jax.numpy as jnp
from jax import lax
from jax.experimental import pallas as pl
from jax.experimental.pallas import tpu as pltpu
from jax.experimental.pallas import tpu_sc as plsc

_VOCAB = 1000000
_EMB = 64
_BATCH = 4096
_SEQ = 200
_CLASSES = 5
_CPAD = 16
_PACK = 8
_VBLK = 8192

_NC = 2
_NS = 16
_NW = _NC * _NS
_ROWS_PER_W = _BATCH // _NW
_CHA = 96
_CHB = _SEQ - _CHA
_NBUF = 4

_mesh = plsc.VectorSubcoreMesh(core_axis_name="c", subcore_axis_name="s")


def _proj_body(tt_ref, w2_ref, o_ref):
    y = tt_ref[...].T.reshape(_VBLK // _PACK, _PACK, _EMB)
    z = jnp.concatenate([y[:, g, :] for g in range(_PACK)], axis=1)
    o_ref[...] = jnp.dot(z, w2_ref[...], preferred_element_type=jnp.float32)


@functools.partial(
    pl.kernel,
    mesh=_mesh,
    out_type=jax.ShapeDtypeStruct((_BATCH, _CPAD), jnp.float32),
    scratch_types=[
        pltpu.VMEM((_ROWS_PER_W, _SEQ), jnp.int32),
        pltpu.VMEM((_NBUF, _SEQ, _CPAD), jnp.float32),
        pltpu.VMEM((_ROWS_PER_W, _CPAD), jnp.float32),
        pltpu.VMEM((_CPAD,), jnp.float32),
        pltpu.SemaphoreType.DMA,
    ],
    compiler_params=pltpu.CompilerParams(use_tc_tiling_on_sc=False),
)
def _sc_pool(proj_hbm, idx_hbm, bias_hbm, out_hbm, idx_v, buf_v, pooled_v,
             bias_v, sem):
    wid = lax.axis_index("s") * _NC + lax.axis_index("c")
    row_base = wid * _ROWS_PER_W

    pltpu.sync_copy(bias_hbm, bias_v)
    pltpu.sync_copy(idx_hbm.at[pl.ds(row_base, _ROWS_PER_W)], idx_v)

    def copies(row, slot):
        return (
            pltpu.make_async_copy(
                proj_hbm.at[idx_v.at[row, pl.ds(0, _CHA)]],
                buf_v.at[slot, pl.ds(0, _CHA)], sem),
            pltpu.make_async_copy(
                proj_hbm.at[idx_v.at[row, pl.ds(_CHA, _CHB)]],
                buf_v.at[slot, pl.ds(_CHA, _CHB)], sem),
        )

    def issue(row, slot):
        for c in copies(row, slot):
            c.start()

    for r in range(_NBUF - 1):
        issue(r, r)

    def row_body(b, _):
        slot = lax.rem(b, _NBUF)

        @pl.when(b + (_NBUF - 1) < _ROWS_PER_W)
        def _():
            issue(b + (_NBUF - 1), lax.rem(b + (_NBUF - 1), _NBUF))

        for c in copies(b, slot):
            c.wait()

        def acc_body(s, acc):
            return acc + buf_v[slot, s, pl.ds(0, _CPAD)]

        acc = lax.fori_loop(0, _SEQ, acc_body, bias_v[pl.ds(0, _CPAD)],
                            unroll=8)
        pooled_v[b, pl.ds(0, _CPAD)] = acc
        return 0

    lax.fori_loop(0, _ROWS_PER_W, row_body, 0)
    pltpu.sync_copy(pooled_v, out_hbm.at[pl.ds(row_base, _ROWS_PER_W)])


def kernel(inputs, emb_table, W, b):
    wt = jnp.pad(W.astype(jnp.float32).T * (1.0 / _SEQ),
                 ((0, 0), (0, _CPAD - _CLASSES)))
    w2 = jnp.kron(jnp.eye(_PACK, dtype=jnp.float32), wt)

    n_packed = _VOCAB // _PACK
    grid = _VOCAB // _VBLK + (1 if _VOCAB % _VBLK else 0)
    proj_packed = pl.pallas_call(
        _proj_body,
        grid=(grid,),
        in_specs=[
            pl.BlockSpec((_EMB, _VBLK), lambda g: (0, g)),
            pl.BlockSpec((_PACK * _EMB, _PACK * _CPAD), lambda g: (0, 0)),
        ],
        out_specs=pl.BlockSpec((_VBLK // _PACK, _PACK * _CPAD),
                               lambda g: (g, 0)),
        out_shape=jax.ShapeDtypeStruct((n_packed, _PACK * _CPAD),
                                       jnp.float32),
    )(emb_table.T, w2)

    proj = proj_packed.reshape(_VOCAB, _CPAD)
    bias_pad = jnp.pad(b.astype(jnp.float32), (0, _CPAD - _CLASSES))
    sums = _sc_pool(proj, inputs, bias_pad)
    return sums[:, :_CLASSES]


# speedup vs baseline: 3.5542x; 1.0598x over previous
import functools

import jax
import jax.numpy as jnp
from jax import lax
from jax.experimental import pallas as pl
from jax.experimental.pallas import tpu as pltpu
from jax.experimental.pallas import tpu_sc as plsc

_VOCAB = 1000000
_EMB = 64
_BATCH = 4096
_SEQ = 200
_CLASSES = 5
_CPAD = 16
_PACK = 8
_VBLK = 8192

_NC = 2
_NS = 16
_NW = _NC * _NS
_ROWS_PER_W = _BATCH // _NW
_CHA = 96
_CHB = _SEQ - _CHA
_NBUF = 8

_mesh = plsc.VectorSubcoreMesh(core_axis_name="c", subcore_axis_name="s")


def _proj_body(tt_ref, w2_ref, o_ref):
    xb = tt_ref[...].astype(jnp.bfloat16)
    y = xb.T.reshape(_VBLK // _PACK, _PACK, _EMB)
    z = jnp.concatenate([y[:, g, :] for g in range(_PACK)], axis=1)
    o_ref[...] = jnp.dot(z, w2_ref[...], preferred_element_type=jnp.float32)


@functools.partial(
    pl.kernel,
    mesh=_mesh,
    out_type=jax.ShapeDtypeStruct((_BATCH, _CPAD), jnp.float32),
    scratch_types=[
        pltpu.VMEM((_ROWS_PER_W, _SEQ), jnp.int32),
        pltpu.VMEM((_NBUF, _SEQ, _CPAD), jnp.float32),
        pltpu.VMEM((_ROWS_PER_W, _CPAD), jnp.float32),
        pltpu.VMEM((_CPAD,), jnp.float32),
        pltpu.SemaphoreType.DMA,
    ],
    compiler_params=pltpu.CompilerParams(use_tc_tiling_on_sc=False),
)
def _sc_pool(proj_hbm, idx_hbm, bias_hbm, out_hbm, idx_v, buf_v, pooled_v,
             bias_v, sem):
    wid = lax.axis_index("s") * _NC + lax.axis_index("c")
    row_base = wid * _ROWS_PER_W

    pltpu.sync_copy(bias_hbm, bias_v)
    pltpu.sync_copy(idx_hbm.at[pl.ds(row_base, _ROWS_PER_W)], idx_v)

    def copies(row, slot):
        return (
            pltpu.make_async_copy(
                proj_hbm.at[idx_v.at[row, pl.ds(0, _CHA)]],
                buf_v.at[slot, pl.ds(0, _CHA)], sem),
            pltpu.make_async_copy(
                proj_hbm.at[idx_v.at[row, pl.ds(_CHA, _CHB)]],
                buf_v.at[slot, pl.ds(_CHA, _CHB)], sem),
        )

    def issue(row, slot):
        for c in copies(row, slot):
            c.start()

    for r in range(_NBUF - 1):
        issue(r, r)

    def row_body(b, _):
        slot = lax.rem(b, _NBUF)

        @pl.when(b + (_NBUF - 1) < _ROWS_PER_W)
        def _():
            issue(b + (_NBUF - 1), lax.rem(b + (_NBUF - 1), _NBUF))

        for c in copies(b, slot):
            c.wait()

        def acc_body(s, acc):
            return acc + buf_v[slot, s, pl.ds(0, _CPAD)]

        acc = lax.fori_loop(0, _SEQ, acc_body, bias_v[pl.ds(0, _CPAD)],
                            unroll=8)
        pooled_v[b, pl.ds(0, _CPAD)] = acc
        return 0

    lax.fori_loop(0, _ROWS_PER_W, row_body, 0)
    pltpu.sync_copy(pooled_v, out_hbm.at[pl.ds(row_base, _ROWS_PER_W)])


def kernel(inputs, emb_table, W, b):
    wt = jnp.pad(W.astype(jnp.float32).T * (1.0 / _SEQ),
                 ((0, 0), (0, _CPAD - _CLASSES)))
    w2 = jnp.kron(jnp.eye(_PACK, dtype=jnp.float32), wt).astype(jnp.bfloat16)

    n_packed = _VOCAB // _PACK
    grid = _VOCAB // _VBLK + (1 if _VOCAB % _VBLK else 0)
    proj_packed = pl.pallas_call(
        _proj_body,
        grid=(grid,),
        in_specs=[
            pl.BlockSpec((_EMB, _VBLK), lambda g: (0, g)),
            pl.BlockSpec((_PACK * _EMB, _PACK * _CPAD), lambda g: (0, 0)),
        ],
        out_specs=pl.BlockSpec((_VBLK // _PACK, _PACK * _CPAD),
                               lambda g: (g, 0)),
        out_shape=jax.ShapeDtypeStruct((n_packed, _PACK * _CPAD),
                                       jnp.float32),
    )(emb_table.T, w2)

    proj = proj_packed.reshape(_VOCAB, _CPAD)
    bias_pad = jnp.pad(b.astype(jnp.float32), (0, _CPAD - _CLASSES))
    sums = _sc_pool(proj, inputs, bias_pad)
    return sums[:, :_CLASSES]


# speedup vs baseline: 5.0297x; 1.4152x over previous
import functools

import jax
import jax.numpy as jnp
from jax import lax
from jax.experimental import pallas as pl
from jax.experimental.pallas import tpu as pltpu
from jax.experimental.pallas import tpu_sc as plsc

_VOCAB = 1000000
_EMB = 64
_BATCH = 4096
_SEQ = 200
_CLASSES = 5
_CPAD = 16
_PACK = 8
_VBLK = 16384

_NC = 2
_NS = 16
_NW = _NC * _NS
_ROWS_PER_W = _BATCH // _NW
_CHA = 96
_CHB = _SEQ - _CHA
_NBUF = 8

_mesh = plsc.VectorSubcoreMesh(core_axis_name="c", subcore_axis_name="s")


def _proj_body(tt_ref, wr_ref, o_ref):
    xb = tt_ref[...].astype(jnp.bfloat16)
    y = lax.dot_general(xb, wr_ref[...], (((0,), (0,)), ((), ())),
                        preferred_element_type=jnp.float32)
    lane = lax.broadcasted_iota(jnp.int32, (_VBLK, _PACK * _CPAD), 1)
    row = lax.broadcasted_iota(jnp.int32, (_VBLK, _PACK * _CPAD), 0)
    m = (lane // _CPAD) == (row % _PACK)
    ym = jnp.where(m, y, 0.0).reshape(_VBLK // _PACK, _PACK, _PACK * _CPAD)
    o_ref[...] = ym.sum(axis=1)


@functools.partial(
    pl.kernel,
    mesh=_mesh,
    out_type=jax.ShapeDtypeStruct((_BATCH, _CPAD), jnp.float32),
    scratch_types=[
        pltpu.VMEM((_ROWS_PER_W, _SEQ), jnp.int32),
        pltpu.VMEM((_NBUF, _SEQ, _CPAD), jnp.float32),
        pltpu.VMEM((_ROWS_PER_W, _CPAD), jnp.float32),
        pltpu.VMEM((_CPAD,), jnp.float32),
        pltpu.SemaphoreType.DMA,
    ],
    compiler_params=pltpu.CompilerParams(use_tc_tiling_on_sc=False),
)
def _sc_pool(proj_hbm, idx_hbm, bias_hbm, out_hbm, idx_v, buf_v, pooled_v,
             bias_v, sem):
    wid = lax.axis_index("s") * _NC + lax.axis_index("c")
    row_base = wid * _ROWS_PER_W

    pltpu.sync_copy(bias_hbm, bias_v)
    pltpu.sync_copy(idx_hbm.at[pl.ds(row_base, _ROWS_PER_W)], idx_v)

    def copies(row, slot):
        return (
            pltpu.make_async_copy(
                proj_hbm.at[idx_v.at[row, pl.ds(0, _CHA)]],
                buf_v.at[slot, pl.ds(0, _CHA)], sem),
            pltpu.make_async_copy(
                proj_hbm.at[idx_v.at[row, pl.ds(_CHA, _CHB)]],
                buf_v.at[slot, pl.ds(_CHA, _CHB)], sem),
        )

    def issue(row, slot):
        for c in copies(row, slot):
            c.start()

    for r in range(_NBUF - 1):
        issue(r, r)

    def row_body(b, _):
        slot = lax.rem(b, _NBUF)

        @pl.when(b + (_NBUF - 1) < _ROWS_PER_W)
        def _():
            issue(b + (_NBUF - 1), lax.rem(b + (_NBUF - 1), _NBUF))

        for c in copies(b, slot):
            c.wait()

        def acc_body(s, acc):
            return acc + buf_v[slot, s, pl.ds(0, _CPAD)]

        acc = lax.fori_loop(0, _SEQ, acc_body, bias_v[pl.ds(0, _CPAD)],
                            unroll=8)
        pooled_v[b, pl.ds(0, _CPAD)] = acc
        return 0

    lax.fori_loop(0, _ROWS_PER_W, row_body, 0)
    pltpu.sync_copy(pooled_v, out_hbm.at[pl.ds(row_base, _ROWS_PER_W)])


def kernel(inputs, emb_table, W, b):
    wt = jnp.pad(W.astype(jnp.float32).T * (1.0 / _SEQ),
                 ((0, 0), (0, _CPAD - _CLASSES)))
    wr = jnp.tile(wt, (1, _PACK)).astype(jnp.bfloat16)

    n_packed = _VOCAB // _PACK
    grid = _VOCAB // _VBLK + (1 if _VOCAB % _VBLK else 0)
    proj_packed = pl.pallas_call(
        _proj_body,
        grid=(grid,),
        in_specs=[
            pl.BlockSpec((_EMB, _VBLK), lambda g: (0, g)),
            pl.BlockSpec((_EMB, _PACK * _CPAD), lambda g: (0, 0)),
        ],
        out_specs=pl.BlockSpec((_VBLK // _PACK, _PACK * _CPAD),
                               lambda g: (g, 0)),
        out_shape=jax.ShapeDtypeStruct((n_packed, _PACK * _CPAD),
                                       jnp.float32),
    )(emb_table.T, wr)

    proj = proj_packed.reshape(_VOCAB, _CPAD)
    bias_pad = jnp.pad(b.astype(jnp.float32), (0, _CPAD - _CLASSES))
    sums = _sc_pool(proj, inputs, bias_pad)
    return sums[:, :_CLASSES]


# speedup vs baseline: 5.1884x; 1.0316x over previous
import functools

import jax
import jax.numpy as jnp
from jax import lax
from jax.experimental import pallas as pl
from jax.experimental.pallas import tpu as pltpu
from jax.experimental.pallas import tpu_sc as plsc

_VOCAB = 1000000
_EMB = 64
_BATCH = 4096
_SEQ = 200
_CLASSES = 5
_CPAD = 16
_PACK = 8
_VBLK = 16384

_NC = 2
_NS = 16
_NW = _NC * _NS
_ROWS_PER_W = _BATCH // _NW
_CHA = 96
_CHB = _SEQ - _CHA
_NBUF = 8

_mesh = plsc.VectorSubcoreMesh(core_axis_name="c", subcore_axis_name="s")


def _proj_body(tt_ref, wr_ref, m_ref, o_ref):
    xb = tt_ref[...].astype(jnp.bfloat16)
    y = lax.dot_general(xb, wr_ref[...], (((0,), (0,)), ((), ())),
                        preferred_element_type=jnp.float32)
    ym = (y.reshape(_VBLK // _PACK, _PACK, _PACK * _CPAD)
          * m_ref[...].reshape(1, _PACK, _PACK * _CPAD))
    o_ref[...] = ym.sum(axis=1)


@functools.partial(
    pl.kernel,
    mesh=_mesh,
    out_type=jax.ShapeDtypeStruct((_BATCH, _CPAD), jnp.float32),
    scratch_types=[
        pltpu.VMEM((_ROWS_PER_W, _SEQ), jnp.int32),
        pltpu.VMEM((_NBUF, _SEQ, _CPAD), jnp.float32),
        pltpu.VMEM((_ROWS_PER_W, _CPAD), jnp.float32),
        pltpu.VMEM((_CPAD,), jnp.float32),
        pltpu.SemaphoreType.DMA,
    ],
    compiler_params=pltpu.CompilerParams(use_tc_tiling_on_sc=False),
)
def _sc_pool(proj_hbm, idx_hbm, bias_hbm, out_hbm, idx_v, buf_v, pooled_v,
             bias_v, sem):
    wid = lax.axis_index("s") * _NC + lax.axis_index("c")
    row_base = wid * _ROWS_PER_W

    pltpu.sync_copy(bias_hbm, bias_v)
    pltpu.sync_copy(idx_hbm.at[pl.ds(row_base, _ROWS_PER_W)], idx_v)

    def copies(row, slot):
        return (
            pltpu.make_async_copy(
                proj_hbm.at[idx_v.at[row, pl.ds(0, _CHA)]],
                buf_v.at[slot, pl.ds(0, _CHA)], sem),
            pltpu.make_async_copy(
                proj_hbm.at[idx_v.at[row, pl.ds(_CHA, _CHB)]],
                buf_v.at[slot, pl.ds(_CHA, _CHB)], sem),
        )

    def issue(row, slot):
        for c in copies(row, slot):
            c.start()

    for r in range(_NBUF - 1):
        issue(r, r)

    def row_body(b, _):
        slot = lax.rem(b, _NBUF)

        @pl.when(b + (_NBUF - 1) < _ROWS_PER_W)
        def _():
            issue(b + (_NBUF - 1), lax.rem(b + (_NBUF - 1), _NBUF))

        for c in copies(b, slot):
            c.wait()

        def acc_body(s, acc):
            return acc + buf_v[slot, s, pl.ds(0, _CPAD)]

        acc = lax.fori_loop(0, _SEQ, acc_body, bias_v[pl.ds(0, _CPAD)],
                            unroll=8)
        pooled_v[b, pl.ds(0, _CPAD)] = acc
        return 0

    lax.fori_loop(0, _ROWS_PER_W, row_body, 0)
    pltpu.sync_copy(pooled_v, out_hbm.at[pl.ds(row_base, _ROWS_PER_W)])


def kernel(inputs, emb_table, W, b):
    wt = jnp.pad(W.astype(jnp.float32).T * (1.0 / _SEQ),
                 ((0, 0), (0, _CPAD - _CLASSES)))
    wr = jnp.tile(wt, (1, _PACK)).astype(jnp.bfloat16)
    lane = lax.broadcasted_iota(jnp.int32, (_PACK, _PACK * _CPAD), 1)
    row = lax.broadcasted_iota(jnp.int32, (_PACK, _PACK * _CPAD), 0)
    m8 = ((lane // _CPAD) == row).astype(jnp.float32)

    n_packed = _VOCAB // _PACK
    grid = _VOCAB // _VBLK + (1 if _VOCAB % _VBLK else 0)
    proj_packed = pl.pallas_call(
        _proj_body,
        grid=(grid,),
        in_specs=[
            pl.BlockSpec((_EMB, _VBLK), lambda g: (0, g)),
            pl.BlockSpec((_EMB, _PACK * _CPAD), lambda g: (0, 0)),
            pl.BlockSpec((_PACK, _PACK * _CPAD), lambda g: (0, 0)),
        ],
        out_specs=pl.BlockSpec((_VBLK // _PACK, _PACK * _CPAD),
                               lambda g: (g, 0)),
        out_shape=jax.ShapeDtypeStruct((n_packed, _PACK * _CPAD),
                                       jnp.float32),
    )(emb_table.T, wr, m8)

    proj = proj_packed.reshape(_VOCAB, _CPAD)
    bias_pad = jnp.pad(b.astype(jnp.float32), (0, _CPAD - _CLASSES))
    sums = _sc_pool(proj, inputs, bias_pad)
    return sums[:, :_CLASSES]


# speedup vs baseline: 5.2414x; 1.0102x over previous
import functools

import jax
import jax.numpy as jnp
from jax import lax
from jax.experimental import pallas as pl
from jax.experimental.pallas import tpu as pltpu
from jax.experimental.pallas import tpu_sc as plsc

_VOCAB = 1000000
_EMB = 64
_BATCH = 4096
_SEQ = 200
_CLASSES = 5
_CPAD = 16
_PACK = 8
_VBLK = 32768

_NC = 2
_NS = 16
_NW = _NC * _NS
_ROWS_PER_W = _BATCH // _NW
_CHA = 96
_CHB = _SEQ - _CHA
_NBUF = 8

_mesh = plsc.VectorSubcoreMesh(core_axis_name="c", subcore_axis_name="s")


def _proj_body(tt_ref, wr_ref, m_ref, o_ref):
    xb = tt_ref[...].astype(jnp.bfloat16)
    y = lax.dot_general(xb, wr_ref[...], (((0,), (0,)), ((), ())),
                        preferred_element_type=jnp.float32)
    ym = (y.reshape(_VBLK // _PACK, _PACK, _PACK * _CPAD)
          * m_ref[...].reshape(1, _PACK, _PACK * _CPAD))
    o_ref[...] = ym.sum(axis=1)


@functools.partial(
    pl.kernel,
    mesh=_mesh,
    out_type=jax.ShapeDtypeStruct((_BATCH, _CPAD), jnp.float32),
    scratch_types=[
        pltpu.VMEM((_ROWS_PER_W, _SEQ), jnp.int32),
        pltpu.VMEM((_NBUF, _SEQ, _CPAD), jnp.float32),
        pltpu.VMEM((_ROWS_PER_W, _CPAD), jnp.float32),
        pltpu.VMEM((_CPAD,), jnp.float32),
        pltpu.SemaphoreType.DMA,
    ],
    compiler_params=pltpu.CompilerParams(use_tc_tiling_on_sc=False),
)
def _sc_pool(proj_hbm, idx_hbm, bias_hbm, out_hbm, idx_v, buf_v, pooled_v,
             bias_v, sem):
    wid = lax.axis_index("s") * _NC + lax.axis_index("c")
    row_base = wid * _ROWS_PER_W

    pltpu.sync_copy(bias_hbm, bias_v)
    pltpu.sync_copy(idx_hbm.at[pl.ds(row_base, _ROWS_PER_W)], idx_v)

    def copies(row, slot):
        return (
            pltpu.make_async_copy(
                proj_hbm.at[idx_v.at[row, pl.ds(0, _CHA)]],
                buf_v.at[slot, pl.ds(0, _CHA)], sem),
            pltpu.make_async_copy(
                proj_hbm.at[idx_v.at[row, pl.ds(_CHA, _CHB)]],
                buf_v.at[slot, pl.ds(_CHA, _CHB)], sem),
        )

    def issue(row, slot):
        for c in copies(row, slot):
            c.start()

    for r in range(_NBUF - 1):
        issue(r, r)

    def row_body(b, _):
        slot = lax.rem(b, _NBUF)

        @pl.when(b + (_NBUF - 1) < _ROWS_PER_W)
        def _():
            issue(b + (_NBUF - 1), lax.rem(b + (_NBUF - 1), _NBUF))

        for c in copies(b, slot):
            c.wait()

        def acc_body(s, acc):
            return acc + buf_v[slot, s, pl.ds(0, _CPAD)]

        acc = lax.fori_loop(0, _SEQ, acc_body, bias_v[pl.ds(0, _CPAD)],
                            unroll=8)
        pooled_v[b, pl.ds(0, _CPAD)] = acc
        return 0

    lax.fori_loop(0, _ROWS_PER_W, row_body, 0)
    pltpu.sync_copy(pooled_v, out_hbm.at[pl.ds(row_base, _ROWS_PER_W)])


def kernel(inputs, emb_table, W, b):
    wt = jnp.pad(W.astype(jnp.float32).T * (1.0 / _SEQ),
                 ((0, 0), (0, _CPAD - _CLASSES)))
    wr = jnp.tile(wt, (1, _PACK)).astype(jnp.bfloat16)
    lane = lax.broadcasted_iota(jnp.int32, (_PACK, _PACK * _CPAD), 1)
    row = lax.broadcasted_iota(jnp.int32, (_PACK, _PACK * _CPAD), 0)
    m8 = ((lane // _CPAD) == row).astype(jnp.float32)

    n_packed = _VOCAB // _PACK
    grid = _VOCAB // _VBLK + (1 if _VOCAB % _VBLK else 0)
    proj_packed = pl.pallas_call(
        _proj_body,
        grid=(grid,),
        in_specs=[
            pl.BlockSpec((_EMB, _VBLK), lambda g: (0, g)),
            pl.BlockSpec((_EMB, _PACK * _CPAD), lambda g: (0, 0)),
            pl.BlockSpec((_PACK, _PACK * _CPAD), lambda g: (0, 0)),
        ],
        out_specs=pl.BlockSpec((_VBLK // _PACK, _PACK * _CPAD),
                               lambda g: (g, 0)),
        out_shape=jax.ShapeDtypeStruct((n_packed, _PACK * _CPAD),
                                       jnp.float32),
    )(emb_table.T, wr, m8)

    proj = proj_packed.reshape(_VOCAB, _CPAD)
    bias_pad = jnp.pad(b.astype(jnp.float32), (0, _CPAD - _CLASSES))
    sums = _sc_pool(proj, inputs, bias_pad)
    return sums[:, :_CLASSES]


# speedup vs baseline: 5.2730x; 1.0060x over previous
import functools

import jax
import jax.numpy as jnp
from jax import lax
from jax.experimental import pallas as pl
from jax.experimental.pallas import tpu as pltpu
from jax.experimental.pallas import tpu_sc as plsc

_VOCAB = 1000000
_EMB = 64
_BATCH = 4096
_SEQ = 200
_CLASSES = 5
_CPAD = 16
_PACK = 8
_VBLK = 32768

_NC = 2
_NS = 16
_NW = _NC * _NS
_ROWS_PER_W = _BATCH // _NW
_CHA = 96
_CHB = _SEQ - _CHA
_NBUF = 16

_mesh = plsc.VectorSubcoreMesh(core_axis_name="c", subcore_axis_name="s")


def _proj_body(tt_ref, wr_ref, m_ref, o_ref):
    xb = tt_ref[...].astype(jnp.bfloat16)
    y = lax.dot_general(xb, wr_ref[...], (((0,), (0,)), ((), ())),
                        preferred_element_type=jnp.float32)
    ym = (y.reshape(_VBLK // _PACK, _PACK, _PACK * _CPAD)
          * m_ref[...].reshape(1, _PACK, _PACK * _CPAD))
    o_ref[...] = ym.sum(axis=1)


@functools.partial(
    pl.kernel,
    mesh=_mesh,
    out_type=jax.ShapeDtypeStruct((_BATCH, _CPAD), jnp.float32),
    scratch_types=[
        pltpu.VMEM((_ROWS_PER_W, _SEQ), jnp.int32),
        pltpu.VMEM((_NBUF, _SEQ, _CPAD), jnp.float32),
        pltpu.VMEM((_ROWS_PER_W, _CPAD), jnp.float32),
        pltpu.VMEM((_CPAD,), jnp.float32),
        pltpu.SemaphoreType.DMA,
    ],
    compiler_params=pltpu.CompilerParams(use_tc_tiling_on_sc=False),
)
def _sc_pool(proj_hbm, idx_hbm, bias_hbm, out_hbm, idx_v, buf_v, pooled_v,
             bias_v, sem):
    wid = lax.axis_index("s") * _NC + lax.axis_index("c")
    row_base = wid * _ROWS_PER_W

    pltpu.sync_copy(bias_hbm, bias_v)
    pltpu.sync_copy(idx_hbm.at[pl.ds(row_base, _ROWS_PER_W)], idx_v)

    def copies(row, slot):
        return (
            pltpu.make_async_copy(
                proj_hbm.at[idx_v.at[row, pl.ds(0, _CHA)]],
                buf_v.at[slot, pl.ds(0, _CHA)], sem),
            pltpu.make_async_copy(
                proj_hbm.at[idx_v.at[row, pl.ds(_CHA, _CHB)]],
                buf_v.at[slot, pl.ds(_CHA, _CHB)], sem),
        )

    def issue(row, slot):
        for c in copies(row, slot):
            c.start()

    for r in range(_NBUF - 1):
        issue(r, r)

    def row_body(b, _):
        slot = lax.rem(b, _NBUF)

        @pl.when(b + (_NBUF - 1) < _ROWS_PER_W)
        def _():
            issue(b + (_NBUF - 1), lax.rem(b + (_NBUF - 1), _NBUF))

        for c in copies(b, slot):
            c.wait()

        def acc_body(s, acc):
            return acc + buf_v[slot, s, pl.ds(0, _CPAD)]

        acc = lax.fori_loop(0, _SEQ, acc_body, bias_v[pl.ds(0, _CPAD)],
                            unroll=8)
        pooled_v[b, pl.ds(0, _CPAD)] = acc
        return 0

    lax.fori_loop(0, _ROWS_PER_W, row_body, 0)
    pltpu.sync_copy(pooled_v, out_hbm.at[pl.ds(row_base, _ROWS_PER_W)])


def kernel(inputs, emb_table, W, b):
    wt = jnp.pad(W.astype(jnp.float32).T * (1.0 / _SEQ),
                 ((0, 0), (0, _CPAD - _CLASSES)))
    wr = jnp.tile(wt, (1, _PACK)).astype(jnp.bfloat16)
    lane = lax.broadcasted_iota(jnp.int32, (_PACK, _PACK * _CPAD), 1)
    row = lax.broadcasted_iota(jnp.int32, (_PACK, _PACK * _CPAD), 0)
    m8 = ((lane // _CPAD) == row).astype(jnp.float32)

    n_packed = _VOCAB // _PACK
    grid = _VOCAB // _VBLK + (1 if _VOCAB % _VBLK else 0)
    proj_packed = pl.pallas_call(
        _proj_body,
        grid=(grid,),
        in_specs=[
            pl.BlockSpec((_EMB, _VBLK), lambda g: (0, g)),
            pl.BlockSpec((_EMB, _PACK * _CPAD), lambda g: (0, 0)),
            pl.BlockSpec((_PACK, _PACK * _CPAD), lambda g: (0, 0)),
        ],
        out_specs=pl.BlockSpec((_VBLK // _PACK, _PACK * _CPAD),
                               lambda g: (g, 0)),
        out_shape=jax.ShapeDtypeStruct((n_packed, _PACK * _CPAD),
                                       jnp.float32),
    )(emb_table.T, wr, m8)

    proj = proj_packed.reshape(_VOCAB, _CPAD)
    bias_pad = jnp.pad(b.astype(jnp.float32), (0, _CPAD - _CLASSES))
    sums = _sc_pool(proj, inputs, bias_pad)
    return sums[:, :_CLASSES]
